# baseline (speedup 1.0000x reference)
import functools

import jax
import jax.numpy as jnp
from jax import lax
from jax.experimental import pallas as pl
from jax.experimental.pallas import tpu as pltpu
from jax.experimental.pallas import tpu_sc as plsc

N, D, E, H_R, H_E = 4096, 1024, 16, 64, 128
TBLK = 1024
HF = E * H_E
KX = HF + E

NW = 32
TOK_W = N // NW
CHUNK = TOK_W * E


def _router_kernel(x_ref, rw1_ref, rb1_ref, rw2_ref, rb2_ref, w_ref):
    xb = x_ref[...]
    hr = jnp.maximum(
        jnp.dot(xb, rw1_ref[...], preferred_element_type=jnp.float32)
        + rb1_ref[...][None, :], 0.0)
    logits = (jnp.dot(hr, rw2_ref[...], preferred_element_type=jnp.float32)
              + rb2_ref[...][None, :])
    logits = logits - jnp.max(logits, axis=-1, keepdims=True)
    ew = jnp.exp(logits)
    w_ref[...] = ew / jnp.sum(ew, axis=-1, keepdims=True)


def _top2_sc_kernel(w_hbm, wtop_hbm, wv, ov):
    wid = lax.axis_index("s") * 2 + lax.axis_index("c")
    base = wid * CHUNK
    pltpu.sync_copy(w_hbm.at[pl.ds(base, CHUNK)], wv)

    idx = lax.iota(jnp.int32, E)
    perms = [jnp.bitwise_xor(idx, k) for k in (1, 2, 4, 8)]

    def gat(v, perm):
        return v.at[perm].get(mode="promise_in_bounds")

    def bfly_argmax(v):
        ix = idx
        for perm in perms:
            pv, pix = gat(v, perm), gat(ix, perm)
            take = (pv > v) | ((pv == v) & (pix < ix))
            v = jnp.where(take, pv, v)
            ix = jnp.where(take, pix, ix)
        return v, ix

    def bfly_sum(v):
        for perm in perms:
            v = v + gat(v, perm)
        return v

    def body(t, carry):
        wrow = wv[pl.ds(t * E, E)]
        _, i1 = bfly_argmax(wrow)
        w2 = jnp.where(idx == i1, -jnp.inf, wrow)
        _, i2 = bfly_argmax(w2)
        mask = (idx == i1) | (idx == i2)
        wt = jnp.where(mask, wrow, 0.0)
        ov[pl.ds(t * E, E)] = wt / (bfly_sum(wt) + 1e-8)
        return carry

    lax.fori_loop(0, TOK_W, body, 0)
    pltpu.sync_copy(ov, wtop_hbm.at[pl.ds(base, CHUNK)])


@functools.partial(
    pl.kernel,
    out_type=jax.ShapeDtypeStruct((N * E,), jnp.float32),
    mesh=plsc.VectorSubcoreMesh(core_axis_name="c", subcore_axis_name="s"),
    scratch_types=[
        pltpu.VMEM((CHUNK,), jnp.float32),
        pltpu.VMEM((CHUNK,), jnp.float32),
    ],
)
def _top2_sc(w_hbm, wtop_hbm, wv, ov):
    _top2_sc_kernel(w_hbm, wtop_hbm, wv, ov)


def _expert_kernel(x_ref, wtop_ref, ew1_ref, eb1_ref, ew2_ref,
                   y_ref, hs_ref):
    xb = x_ref[...].astype(jnp.bfloat16)
    pre = (jnp.dot(xb, ew1_ref[...], preferred_element_type=jnp.float32)
           + eb1_ref[...])
    h = jnp.tanh(pre)
    wt = wtop_ref[...]
    gates = jnp.broadcast_to(wt[:, :, None], (TBLK, E, H_E)).reshape(TBLK, HF)
    hs_ref[:, :HF] = (h * gates).astype(jnp.bfloat16)
    hs_ref[:, HF:] = wt.astype(jnp.bfloat16)
    y_ref[...] = jnp.dot(hs_ref[...], ew2_ref[...],
                         preferred_element_type=jnp.float32)


@jax.jit
def kernel(x, rw1, rb1, rw2, rb2, ew1, eb1, ew2, eb2):
    w = pl.pallas_call(
        _router_kernel,
        grid=(1,),
        in_specs=[
            pl.BlockSpec((N, D), lambda i: (0, 0)),
            pl.BlockSpec((D, H_R), lambda i: (0, 0)),
            pl.BlockSpec((H_R,), lambda i: (0,)),
            pl.BlockSpec((H_R, E), lambda i: (0, 0)),
            pl.BlockSpec((E,), lambda i: (0,)),
        ],
        out_specs=pl.BlockSpec((N, E), lambda i: (0, 0)),
        out_shape=jax.ShapeDtypeStruct((N, E), jnp.float32),
    )(x, rw1, rb1, rw2, rb2)

    wtop = _top2_sc(w.reshape(N * E)).reshape(N, E)

    ew1f = ew1.transpose(1, 0, 2).reshape(D, HF).astype(jnp.bfloat16)
    eb1f = eb1.reshape(1, HF)
    ew2f = jnp.concatenate(
        [ew2.reshape(HF, D), eb2], axis=0).astype(jnp.bfloat16)

    y = pl.pallas_call(
        _expert_kernel,
        grid=(N // TBLK,),
        in_specs=[
            pl.BlockSpec((TBLK, D), lambda i: (i, 0)),
            pl.BlockSpec((TBLK, E), lambda i: (i, 0)),
            pl.BlockSpec((D, HF), lambda i: (0, 0)),
            pl.BlockSpec((1, HF), lambda i: (0, 0)),
            pl.BlockSpec((KX, D), lambda i: (0, 0)),
        ],
        out_specs=pl.BlockSpec((TBLK, D), lambda i: (i, 0)),
        out_shape=jax.ShapeDtypeStruct((N, D), jnp.float32),
        scratch_shapes=[pltpu.VMEM((TBLK, KX), jnp.bfloat16)],
        compiler_params=pltpu.CompilerParams(
            dimension_semantics=("parallel",)),
    )(x, wtop, ew1f, eb1f, ew2f)
    return (y, w)


# speedup vs baseline: 1.2385x; 1.2385x over previous
import jax
import jax.numpy as jnp
from jax.experimental import pallas as pl
from jax.experimental.pallas import tpu as pltpu

N, D, E, H_R, H_E = 4096, 1024, 16, 64, 128
TBLK = 1024
HF = E * H_E
KX = HF + E


def _router_kernel(x_ref, rw1_ref, rb1_ref, rw2_ref, rb2_ref,
                   w_ref, wtop_ref):
    xb = x_ref[...]
    hr = jnp.maximum(
        jnp.dot(xb, rw1_ref[...], preferred_element_type=jnp.float32)
        + rb1_ref[...][None, :], 0.0)
    logits = (jnp.dot(hr, rw2_ref[...], preferred_element_type=jnp.float32)
              + rb2_ref[...][None, :])
    logits = logits - jnp.max(logits, axis=-1, keepdims=True)
    ew = jnp.exp(logits)
    w = ew / jnp.sum(ew, axis=-1, keepdims=True)
    w_ref[...] = w
    cols = jax.lax.broadcasted_iota(jnp.int32, w.shape, 1)
    i1 = jnp.argmax(w, axis=-1)[:, None]
    w2 = jnp.where(cols == i1, -jnp.inf, w)
    i2 = jnp.argmax(w2, axis=-1)[:, None]
    mask = (cols == i1) | (cols == i2)
    wt = jnp.where(mask, w, 0.0)
    wtop_ref[...] = wt / (jnp.sum(wt, axis=-1, keepdims=True) + 1e-8)


def _expert_kernel(x_ref, wtop_ref, ew1_ref, eb1_ref, ew2_ref,
                   y_ref, hs_ref):
    xb = x_ref[...].astype(jnp.bfloat16)
    pre = (jnp.dot(xb, ew1_ref[...], preferred_element_type=jnp.float32)
           + eb1_ref[...])
    h = jnp.tanh(pre)
    wt = wtop_ref[...]
    gates = jnp.broadcast_to(wt[:, :, None], (TBLK, E, H_E)).reshape(TBLK, HF)
    hs_ref[:, :HF] = (h * gates).astype(jnp.bfloat16)
    hs_ref[:, HF:] = wt.astype(jnp.bfloat16)
    y_ref[...] = jnp.dot(hs_ref[...], ew2_ref[...],
                         preferred_element_type=jnp.float32)


@jax.jit
def kernel(x, rw1, rb1, rw2, rb2, ew1, eb1, ew2, eb2):
    w, wtop = pl.pallas_call(
        _router_kernel,
        grid=(1,),
        in_specs=[
            pl.BlockSpec((N, D), lambda i: (0, 0)),
            pl.BlockSpec((D, H_R), lambda i: (0, 0)),
            pl.BlockSpec((H_R,), lambda i: (0,)),
            pl.BlockSpec((H_R, E), lambda i: (0, 0)),
            pl.BlockSpec((E,), lambda i: (0,)),
        ],
        out_specs=[
            pl.BlockSpec((N, E), lambda i: (0, 0)),
            pl.BlockSpec((N, E), lambda i: (0, 0)),
        ],
        out_shape=[
            jax.ShapeDtypeStruct((N, E), jnp.float32),
            jax.ShapeDtypeStruct((N, E), jnp.float32),
        ],
    )(x, rw1, rb1, rw2, rb2)

    ew1f = ew1.transpose(1, 0, 2).reshape(D, HF).astype(jnp.bfloat16)
    eb1f = eb1.reshape(1, HF)
    ew2f = jnp.concatenate(
        [ew2.reshape(HF, D), eb2], axis=0).astype(jnp.bfloat16)

    y = pl.pallas_call(
        _expert_kernel,
        grid=(N // TBLK,),
        in_specs=[
            pl.BlockSpec((TBLK, D), lambda i: (i, 0)),
            pl.BlockSpec((TBLK, E), lambda i: (i, 0)),
            pl.BlockSpec((D, HF), lambda i: (0, 0)),
            pl.BlockSpec((1, HF), lambda i: (0, 0)),
            pl.BlockSpec((KX, D), lambda i: (0, 0)),
        ],
        out_specs=pl.BlockSpec((TBLK, D), lambda i: (i, 0)),
        out_shape=jax.ShapeDtypeStruct((N, D), jnp.float32),
        scratch_shapes=[pltpu.VMEM((TBLK, KX), jnp.bfloat16)],
        compiler_params=pltpu.CompilerParams(
            dimension_semantics=("parallel",)),
    )(x, wtop, ew1f, eb1f, ew2f)
    return (y, w)
